# speedup vs baseline: 1.8006x; 1.8006x over previous
import functools

import jax
import jax.numpy as jnp
from jax import lax
from jax.experimental import pallas as pl
from jax.experimental.pallas import tpu as pltpu
from jax.experimental.pallas import tpu_sc as plsc

DIM = 128
BATCH = 16384
NC = 2
NS = 16
L = 16
NW = NC * NS
BPW = BATCH // NW
CH = 128
NCHUNK = BPW // CH
VPR = DIM // L

_mesh = plsc.VectorSubcoreMesh(
    core_axis_name="c", subcore_axis_name="s", num_cores=NC, num_subcores=NS
)


@functools.partial(
    pl.kernel,
    out_type=jax.ShapeDtypeStruct((NW, L), jnp.float32),
    mesh=_mesh,
    scratch_types=[
        pltpu.VMEM((3 * NCHUNK, CH), jnp.int32),
        pltpu.VMEM((2, 3, CH, DIM), jnp.float32),
        pltpu.VMEM((L,), jnp.float32),
        pltpu.SemaphoreType.DMA,
        pltpu.SemaphoreType.DMA,
    ],
)
def _transe_loss_sc(ins_hbm, rel_hbm, idx_hbm, out_hbm, idx_v, rows_v, acc_v,
                    sem0, sem1):
    wid = lax.axis_index("s") * NC + lax.axis_index("c")
    sems = (sem0, sem1)

    pltpu.sync_copy(idx_hbm.at[wid], idx_v)

    def fire(c, buf):
        sem = sems[buf]
        pltpu.async_copy(ins_hbm.at[idx_v.at[0 * NCHUNK + c]],
                         rows_v.at[buf, 0], sem)
        pltpu.async_copy(rel_hbm.at[idx_v.at[1 * NCHUNK + c]],
                         rows_v.at[buf, 1], sem)
        pltpu.async_copy(ins_hbm.at[idx_v.at[2 * NCHUNK + c]],
                         rows_v.at[buf, 2], sem)

    def drain(c, buf):
        sem = sems[buf]
        pltpu.make_async_copy(ins_hbm.at[idx_v.at[0 * NCHUNK + c]],
                              rows_v.at[buf, 0], sem).wait()
        pltpu.make_async_copy(rel_hbm.at[idx_v.at[1 * NCHUNK + c]],
                              rows_v.at[buf, 1], sem).wait()
        pltpu.make_async_copy(ins_hbm.at[idx_v.at[2 * NCHUNK + c]],
                              rows_v.at[buf, 2], sem).wait()

    fire(0, 0)

    accs = tuple(jnp.zeros((L,), jnp.float32) for _ in range(VPR))
    for c in range(NCHUNK):
        buf = c % 2
        if c + 1 < NCHUNK:
            fire(c + 1, (c + 1) % 2)
        drain(c, buf)

        def body(s, accs):
            new = []
            for k in range(VPR):
                sl = pl.ds(k * L, L)
                e1 = rows_v[buf, 0, s, sl]
                r = rows_v[buf, 1, s, sl]
                e2 = rows_v[buf, 2, s, sl]
                d = (e1 + r) - e2
                new.append(accs[k] + d * d)
            return tuple(new)

        accs = lax.fori_loop(0, CH, body, accs)

    total = accs[0]
    for k in range(1, VPR):
        total = total + accs[k]
    acc_v[...] = total
    pltpu.sync_copy(acc_v, out_hbm.at[wid])


def kernel(ins_emb, rel_emb, sample):
    idx = sample.astype(jnp.int32).T
    idx = idx.reshape(3, NW, NCHUNK, CH)
    idx = jnp.transpose(idx, (1, 0, 2, 3)).reshape(NW, 3 * NCHUNK, CH)
    partials = _transe_loss_sc(ins_emb, rel_emb, idx)
    return jnp.sum(partials)
